# baseline (speedup 1.0000x reference)
import functools

import jax
import jax.numpy as jnp
from jax import lax
from jax.experimental import pallas as pl
from jax.experimental.pallas import tpu as pltpu

DIM = 64
CAP = 100000
BATCH = 1024
K = 50
DELTA = 0.001

BC = 2048
CAP_PAD = ((CAP + BC - 1) // BC) * BC


def _d2_block_kernel(q_ref, qsq_ref, kt_ref, ksq_ref, out_ref):
    qk = jax.lax.dot_general(
        q_ref[...], kt_ref[...],
        dimension_numbers=(((1,), (1,)), ((), ())),
        preferred_element_type=jnp.float32,
    )
    out_ref[...] = (qsq_ref[...] - 2.0 * qk) + ksq_ref[...]


def _d2_matrix(q, qsq, kt_pad, ksq_pad):
    grid = (CAP_PAD // BC,)
    return pl.pallas_call(
        _d2_block_kernel,
        grid=grid,
        in_specs=[
            pl.BlockSpec((BATCH, DIM), lambda i: (0, 0)),
            pl.BlockSpec((BATCH, 1), lambda i: (0, 0)),
            pl.BlockSpec((BC, DIM), lambda i: (i, 0)),
            pl.BlockSpec((1, BC), lambda i: (0, i)),
        ],
        out_specs=pl.BlockSpec((BATCH, BC), lambda i: (0, i)),
        out_shape=jax.ShapeDtypeStruct((BATCH, CAP_PAD), jnp.float32),
    )(q, qsq, kt_pad, ksq_pad)


def kernel(key, keys_table, values_table):
    qsq = jnp.sum(key * key, axis=1, keepdims=True)
    kt_pad = jnp.pad(keys_table, ((0, CAP_PAD - CAP), (0, 0)))
    ksq_pad = jnp.pad(jnp.sum(keys_table * keys_table, axis=1),
                      (0, CAP_PAD - CAP))[None, :]
    d2 = _d2_matrix(key, qsq, kt_pad, ksq_pad)
    d2 = d2[:, :CAP]
    neg_scores, idx = jax.lax.top_k(-d2, K)
    retrieved_keys = jnp.take(keys_table, idx, axis=0)
    retrieved_values = jnp.take(values_table, idx, axis=0)
    diff = key[:, None, :] - retrieved_keys
    weights = 1.0 / (jnp.sum(diff * diff, axis=-1) + DELTA)
    weights_total = jnp.sum(weights, axis=-1, keepdims=True)
    output_value = jnp.sum(weights * retrieved_values, axis=-1, keepdims=True)
    output_value = output_value / weights_total
    retrieved_distances = -neg_scores
    return output_value, idx, retrieved_distances


# speedup vs baseline: 5.3953x; 5.3953x over previous
import functools

import numpy as np
import jax
import jax.numpy as jnp
from jax import lax
from jax.experimental import pallas as pl
from jax.experimental.pallas import tpu as pltpu
from jax.experimental.pallas import tpu_sc as plsc

DIM = 64
CAP = 100000
BATCH = 1024
K = 50
DELTA = 0.001

BC = 2048
CAP_PAD = ((CAP + BC - 1) // BC) * BC

L = 16
NWORKERS = 32
TOPW = 64
CC = 544
TRIG = CC - TOPW
INF = np.float32(3e38)
IMAX = np.int32(0x7FFFFFFF)


def _d2_block_kernel(q_ref, qsq_ref, kt_ref, ksq_ref, out_ref):
    qk = jax.lax.dot_general(
        q_ref[...], kt_ref[...],
        dimension_numbers=(((1,), (1,)), ((), ())),
        preferred_element_type=jnp.float32,
    )
    out_ref[...] = (qsq_ref[...] - 2.0 * qk) + ksq_ref[...]


def _d2_matrix(q, qsq, kt_pad, ksq_pad):
    grid = (CAP_PAD // BC,)
    return pl.pallas_call(
        _d2_block_kernel,
        grid=grid,
        in_specs=[
            pl.BlockSpec((BATCH, DIM), lambda i: (0, 0)),
            pl.BlockSpec((BATCH, 1), lambda i: (0, 0)),
            pl.BlockSpec((BC, DIM), lambda i: (i, 0)),
            pl.BlockSpec((1, BC), lambda i: (0, i)),
        ],
        out_specs=pl.BlockSpec((BATCH, BC), lambda i: (0, i)),
        out_shape=jax.ShapeDtypeStruct((BATCH, CAP_PAD), jnp.float32),
    )(q, qsq, kt_pad, ksq_pad)


def _merge2(av, ai, bv, bi):
    rv = lax.rev(bv, (0,))
    ri = lax.rev(bi, (0,))
    m = rv < av
    lov = jnp.where(m, rv, av)
    loi = jnp.where(m, ri, ai)
    hiv = jnp.where(m, av, rv)
    hii = jnp.where(m, ai, ri)
    lov, loi = plsc.sort_key_val(lov, loi)
    hiv, hii = plsc.sort_key_val(hiv, hii)
    return lov, loi, hiv, hii


def _insert(ts, sv, si):
    t0v, t0i, t1v, t1i, t2v, t2i, t3v, t3i = ts
    t0v, t0i, sv, si = _merge2(t0v, t0i, sv, si)
    t1v, t1i, sv, si = _merge2(t1v, t1i, sv, si)
    t2v, t2i, sv, si = _merge2(t2v, t2i, sv, si)
    t3v, t3i, sv, si = _merge2(t3v, t3i, sv, si)
    return (t0v, t0i, t1v, t1i, t2v, t2i, t3v, t3i)


def _compact(cval, cidx, count):
    io = lax.iota(jnp.int32, L)
    init = (jnp.full((L,), INF, jnp.float32), jnp.full((L,), IMAX, jnp.int32)) * 4

    def body(j, ts):
        base = j * L
        v = cval[pl.ds(base, L)]
        i = cidx[pl.ds(base, L)]
        valid = (base + io) < count
        v = jnp.where(valid, v, INF)
        i = jnp.where(valid, i, IMAX)
        sv, si = plsc.sort_key_val(v, i)
        return _insert(ts, sv, si)

    return lax.fori_loop(0, CC // L, body, init, unroll=False)


def _append(cval, cidx, count, d, idxv, m):
    n = _popcount(m)
    plsc.store_compressed(cval.at[pl.ds(count, L)], d, mask=m)
    plsc.store_compressed(cidx.at[pl.ds(count, L)], idxv, mask=m)
    return count + n


def _gather16(v, idx):
    dn = lax.GatherDimensionNumbers(offset_dims=(), collapsed_slice_dims=(0,),
                                    start_index_map=(0,))
    return lax.gather(v, idx[:, None], dn, (1,),
                      mode=lax.GatherScatterMode.PROMISE_IN_BOUNDS)


def _shift_up(v, boundary, io):
    g = _gather16(v, jnp.maximum(io - 1, 0))
    return jnp.where(io == 0, boundary, g)


def _shift_down(v, boundary, io):
    g = _gather16(v, jnp.minimum(io + 1, L - 1))
    return jnp.where(io == L - 1, boundary, g)


def _lane(v, lane):
    return v[lane]


def _popcount(m):
    return plsc.all_reduce_population_count(m)[0]


def _hsum16v(v, io):
    for sh in (8, 4, 2, 1):
        v = v + _gather16(v, (io + sh) & (L - 1))
    return v


def _tie_fix(ts):
    io = lax.iota(jnp.int32, L)
    lane_par = io % 2
    vs = list(ts)
    for p in (0, 1, 0, 1):
        t0v, t0i, t1v, t1i, t2v, t2i, t3v, t3i = vs
        tv = [t0v, t1v, t2v, t3v]
        ti = [t0i, t1i, t2i, t3i]
        new_i = []
        for j in range(4):
            pv_b = jnp.float32(-INF) if j == 0 else _lane(tv[j - 1], L - 1)
            pi_b = IMAX if j == 0 else _lane(ti[j - 1], L - 1)
            nv_b = jnp.float32(INF) if j == 3 else _lane(tv[j + 1], 0)
            ni_b = IMAX if j == 3 else _lane(ti[j + 1], 0)
            pv = _shift_up(tv[j], pv_b, io)
            pi = _shift_up(ti[j], pi_b, io)
            nv = _shift_down(tv[j], nv_b, io)
            ni = _shift_down(ti[j], ni_b, io)
            swap_prev = (tv[j] == pv) & (ti[j] < pi) & ((1 - lane_par) == p)
            swap_next = (nv == tv[j]) & (ni < ti[j]) & (lane_par == p)
            new_i.append(jnp.where(swap_prev, pi, jnp.where(swap_next, ni, ti[j])))
        vs = [t0v, new_i[0], t1v, new_i[1], t2v, new_i[2], t3v, new_i[3]]
    return tuple(vs)


def _sc_topk_combine(batch, cap_pad, chunk, nworkers, nq, cap_real,
                     interpret=False):
    nchunks = cap_pad // chunk
    nv_u = chunk // (L * 4)
    mesh = plsc.VectorSubcoreMesh(core_axis_name="c", subcore_axis_name="s",
                                  num_cores=2, num_subcores=16)

    def body(d2_hbm, key_hbm, aug_hbm,
             oval_hbm, oidx_hbm, odist_hbm,
             buf0, buf1, qkeys, cval, cidx, rows, sidx, sval, ovalbuf,
             sem0, sem1, semg):
        io = lax.iota(jnp.int32, L)
        wid = lax.axis_index("s") * 2 + lax.axis_index("c")
        qbase = wid * nq
        bufs = (buf0, buf1)
        sems = (sem0, sem1)

        pltpu.sync_copy(key_hbm.at[pl.ds(qbase, nq)], qkeys)
        pltpu.async_copy(d2_hbm.at[qbase, pl.ds(0, chunk)], buf0, sem0)

        def scan_chunk(buf, chunk_base, count, thr):
            def it(i, carry):
                cnt, th = carry
                base = i * (L * 4)
                d = [buf[pl.ds(base + k * L, L)] for k in range(4)]
                ms = [dk <= th for dk in d]
                anym = (ms[0] | ms[1]) | (ms[2] | ms[3])

                def app(c):
                    cnt2, th2 = c
                    for k in range(4):
                        idxv = io + (chunk_base + base + k * L)
                        cnt2 = _append(cval, cidx, cnt2, d[k], idxv, ms[k])

                    def do_comp(cth):
                        ts = _compact(cval, cidx, cth[0])
                        for j in range(4):
                            cval[pl.ds(j * L, L)] = ts[2 * j]
                            cidx[pl.ds(j * L, L)] = ts[2 * j + 1]
                        return (jnp.int32(TOPW), _lane(ts[6], L - 1))

                    return lax.cond(cnt2 > TRIG, do_comp, lambda cth: cth,
                                    (cnt2, th2))

                return lax.cond(_popcount(anym) > 0, app, lambda c: c,
                                (cnt, th))

            return lax.fori_loop(0, nv_u, it, (count, thr), unroll=False)

        def per_query(qi, carry):
            row = qbase + qi
            count = jnp.int32(0)
            thr = INF
            for c in range(nchunks):
                buf = bufs[c % 2]
                pltpu.make_async_copy(
                    d2_hbm.at[row, pl.ds(c * chunk, chunk)], buf,
                    sems[c % 2]).wait()
                if c < nchunks - 1:
                    pltpu.async_copy(
                        d2_hbm.at[row, pl.ds((c + 1) * chunk, chunk)],
                        bufs[(c + 1) % 2], sems[(c + 1) % 2])
                else:
                    @pl.when(qi + 1 < nq)
                    def _():
                        pltpu.async_copy(
                            d2_hbm.at[row + 1, pl.ds(0, chunk)], bufs[0],
                            sems[0])
                count, thr = scan_chunk(buf, c * chunk, count, thr)

            ts = _compact(cval, cidx, count)
            ts = _tie_fix(ts)
            for j in range(4):
                sval[pl.ds(j * L, L)] = ts[2 * j]
                sidx[pl.ds(j * L, L)] = ts[2 * j + 1]
            pltpu.sync_copy(sval, odist_hbm.at[row])
            pltpu.sync_copy(sidx, oidx_hbm.at[row])

            pltpu.async_copy(aug_hbm.at[sidx], rows, semg).wait()

            qk = [qkeys[qi, pl.ds(k * L, L)] for k in range(4)]

            def wbody(r, c):
                wsumv, vsumv = c
                acc = jnp.zeros((L,), jnp.float32)
                for k in range(4):
                    dk = rows[r, pl.ds(k * L, L)] - qk[k]
                    acc = acc + dk * dk
                wv = 1.0 / (_hsum16v(acc, io) + jnp.float32(DELTA))
                valv = _gather16(rows[r, pl.ds(DIM, L)], io * 0)
                return (wsumv + wv, vsumv + wv * valv)

            zv = jnp.zeros((L,), jnp.float32)
            wsumv, vsumv = lax.fori_loop(0, K, wbody, (zv, zv), unroll=False)
            oval = vsumv / wsumv
            nh = nq // L if nq >= L else 1
            for h in range(nh):
                @pl.when((qi // L) == h)
                def _():
                    cur = ovalbuf[pl.ds(h * L, L)]
                    ovalbuf[pl.ds(h * L, L)] = jnp.where(
                        io == (qi - h * L), oval, cur)
            return carry

        lax.fori_loop(0, nq, per_query, None, unroll=False)
        pltpu.sync_copy(ovalbuf, oval_hbm.at[pl.ds(qbase, nq)])

    return pl.kernel(
        body,
        out_type=[
            jax.ShapeDtypeStruct((batch,), jnp.float32),
            jax.ShapeDtypeStruct((batch, TOPW), jnp.int32),
            jax.ShapeDtypeStruct((batch, TOPW), jnp.float32),
        ],
        mesh=mesh,
        scratch_types=[
            pltpu.VMEM((chunk,), jnp.float32),
            pltpu.VMEM((chunk,), jnp.float32),
            pltpu.VMEM((nq, DIM), jnp.float32),
            pltpu.VMEM((CC,), jnp.float32),
            pltpu.VMEM((CC,), jnp.int32),
            pltpu.VMEM((TOPW, 2 * DIM), jnp.float32),
            pltpu.VMEM((TOPW,), jnp.int32),
            pltpu.VMEM((TOPW,), jnp.float32),
            pltpu.VMEM((max(nq, L),), jnp.float32),
            pltpu.SemaphoreType.DMA,
            pltpu.SemaphoreType.DMA,
            pltpu.SemaphoreType.DMA,
        ],
        compiler_params=pltpu.CompilerParams(needs_layout_passes=False),
        interpret=interpret,
    )


def kernel(key, keys_table, values_table):
    qsq = jnp.sum(key * key, axis=1, keepdims=True)
    kt_pad = jnp.pad(keys_table, ((0, CAP_PAD - CAP), (0, 0)))
    ksq_pad = jnp.pad(jnp.sum(keys_table * keys_table, axis=1),
                      (0, CAP_PAD - CAP), constant_values=1e30)[None, :]
    d2 = _d2_matrix(key, qsq, kt_pad, ksq_pad)
    aug = jnp.concatenate(
        [keys_table, values_table[:, None],
         jnp.zeros((CAP, 2 * DIM - DIM - 1), jnp.float32)], axis=1)
    nq = BATCH // NWORKERS
    sc = _sc_topk_combine(BATCH, CAP_PAD, CAP_PAD // 4, NWORKERS, nq, CAP)
    oval, oidx, odist = sc(d2, key, aug)
    return oval[:, None], oidx[:, :K], odist[:, :K]


# speedup vs baseline: 11.2225x; 2.0800x over previous
import functools

import numpy as np
import jax
import jax.numpy as jnp
from jax import lax
from jax.experimental import pallas as pl
from jax.experimental.pallas import tpu as pltpu
from jax.experimental.pallas import tpu_sc as plsc

DIM = 64
CAP = 100000
BATCH = 1024
K = 50
DELTA = 0.001

BC = 2048
CAP_PAD = ((CAP + BC - 1) // BC) * BC

L = 16
NWORKERS = 32
TOPW = 64
G = 16
CC = 672
TRIG = CC - G * L
INF = np.float32(3e38)
IMAX = np.int32(0x7FFFFFFF)


def _d2_block_kernel(q_ref, qsq_ref, kt_ref, ksq_ref, out_ref):
    qk = jax.lax.dot_general(
        q_ref[...], kt_ref[...],
        dimension_numbers=(((1,), (1,)), ((), ())),
        preferred_element_type=jnp.float32,
    )
    out_ref[...] = (qsq_ref[...] - 2.0 * qk) + ksq_ref[...]


def _d2_matrix(q, qsq, kt_pad, ksq_pad):
    grid = (CAP_PAD // BC,)
    return pl.pallas_call(
        _d2_block_kernel,
        grid=grid,
        in_specs=[
            pl.BlockSpec((BATCH, DIM), lambda i: (0, 0)),
            pl.BlockSpec((BATCH, 1), lambda i: (0, 0)),
            pl.BlockSpec((BC, DIM), lambda i: (i, 0)),
            pl.BlockSpec((1, BC), lambda i: (0, i)),
        ],
        out_specs=pl.BlockSpec((BATCH, BC), lambda i: (0, i)),
        out_shape=jax.ShapeDtypeStruct((BATCH, CAP_PAD), jnp.float32),
    )(q, qsq, kt_pad, ksq_pad)


def _merge2(av, ai, bv, bi):
    rv = lax.rev(bv, (0,))
    ri = lax.rev(bi, (0,))
    m = rv < av
    lov = jnp.where(m, rv, av)
    loi = jnp.where(m, ri, ai)
    hiv = jnp.where(m, av, rv)
    hii = jnp.where(m, ai, ri)
    lov, loi = plsc.sort_key_val(lov, loi)
    hiv, hii = plsc.sort_key_val(hiv, hii)
    return lov, loi, hiv, hii


def _insert(ts, sv, si):
    t0v, t0i, t1v, t1i, t2v, t2i, t3v, t3i = ts
    t0v, t0i, sv, si = _merge2(t0v, t0i, sv, si)
    t1v, t1i, sv, si = _merge2(t1v, t1i, sv, si)
    t2v, t2i, sv, si = _merge2(t2v, t2i, sv, si)
    t3v, t3i, sv, si = _merge2(t3v, t3i, sv, si)
    return (t0v, t0i, t1v, t1i, t2v, t2i, t3v, t3i)


def _compact(cval, cidx, count):
    io = lax.iota(jnp.int32, L)
    init = (jnp.full((L,), INF, jnp.float32), jnp.full((L,), IMAX, jnp.int32)) * 4

    def body(j, ts):
        base = j * L
        v = cval[pl.ds(base, L)]
        i = cidx[pl.ds(base, L)]
        valid = (base + io) < count
        v = jnp.where(valid, v, INF)
        i = jnp.where(valid, i, IMAX)
        sv, si = plsc.sort_key_val(v, i)
        return _insert(ts, sv, si)

    return lax.fori_loop(0, CC // L, body, init, unroll=False)


def _append(cval, cidx, count, d, idxv, m):
    n = _popcount(m)
    plsc.store_compressed(cval.at[pl.ds(count, L)], d, mask=m)
    plsc.store_compressed(cidx.at[pl.ds(count, L)], idxv, mask=m)
    return count + n


def _gather16(v, idx):
    dn = lax.GatherDimensionNumbers(offset_dims=(), collapsed_slice_dims=(0,),
                                    start_index_map=(0,))
    return lax.gather(v, idx[:, None], dn, (1,),
                      mode=lax.GatherScatterMode.PROMISE_IN_BOUNDS)


def _shift_up(v, boundary, io):
    g = _gather16(v, jnp.maximum(io - 1, 0))
    return jnp.where(io == 0, boundary, g)


def _shift_down(v, boundary, io):
    g = _gather16(v, jnp.minimum(io + 1, L - 1))
    return jnp.where(io == L - 1, boundary, g)


def _lane(v, lane):
    return v[lane]


def _popcount(m):
    return plsc.all_reduce_population_count(m)[0]


def _hsum16v(v, io):
    for sh in (8, 4, 2, 1):
        v = v + _gather16(v, (io + sh) & (L - 1))
    return v


def _tie_fix(ts):
    io = lax.iota(jnp.int32, L)
    lane_par = io % 2
    vs = list(ts)
    for p in (0, 1, 0, 1):
        t0v, t0i, t1v, t1i, t2v, t2i, t3v, t3i = vs
        tv = [t0v, t1v, t2v, t3v]
        ti = [t0i, t1i, t2i, t3i]
        new_i = []
        for j in range(4):
            pv_b = jnp.float32(-INF) if j == 0 else _lane(tv[j - 1], L - 1)
            pi_b = IMAX if j == 0 else _lane(ti[j - 1], L - 1)
            nv_b = jnp.float32(INF) if j == 3 else _lane(tv[j + 1], 0)
            ni_b = IMAX if j == 3 else _lane(ti[j + 1], 0)
            pv = _shift_up(tv[j], pv_b, io)
            pi = _shift_up(ti[j], pi_b, io)
            nv = _shift_down(tv[j], nv_b, io)
            ni = _shift_down(ti[j], ni_b, io)
            swap_prev = (tv[j] == pv) & (ti[j] < pi) & ((1 - lane_par) == p)
            swap_next = (nv == tv[j]) & (ni < ti[j]) & (lane_par == p)
            new_i.append(jnp.where(swap_prev, pi, jnp.where(swap_next, ni, ti[j])))
        vs = [t0v, new_i[0], t1v, new_i[1], t2v, new_i[2], t3v, new_i[3]]
    return tuple(vs)


def _sc_topk_combine(batch, cap_pad, chunk, nworkers, nq, cap_real,
                     interpret=False):
    nchunks = cap_pad // chunk
    nv_u = chunk // (L * G)
    mesh = plsc.VectorSubcoreMesh(core_axis_name="c", subcore_axis_name="s",
                                  num_cores=2, num_subcores=16)

    def body(d2_hbm, key_hbm, aug_hbm,
             oval_hbm, oidx_hbm, odist_hbm,
             buf0, buf1, qkeys, cval, cidx, rows, sidx, sval, ovalbuf,
             sem0, sem1, semg):
        io = lax.iota(jnp.int32, L)
        wid = lax.axis_index("s") * 2 + lax.axis_index("c")
        qbase = wid * nq
        bufs = (buf0, buf1)
        sems = (sem0, sem1)

        pltpu.sync_copy(key_hbm.at[pl.ds(qbase, nq)], qkeys)
        pltpu.async_copy(d2_hbm.at[qbase, pl.ds(0, chunk)], buf0, sem0)

        def write_back(ts):
            for j in range(4):
                cval[pl.ds(j * L, L)] = ts[2 * j]
                cidx[pl.ds(j * L, L)] = ts[2 * j + 1]

        def group_append(c, d, ms, gbase):
            cnt2, th2 = c
            for k in range(G):
                idxv = io + (gbase + k * L)
                cnt2 = _append(cval, cidx, cnt2, d[k], idxv, ms[k])

            def do_comp(cth):
                ts = _compact(cval, cidx, cth[0])
                write_back(ts)
                return (jnp.int32(TOPW), _lane(ts[6], 1))

            return lax.cond(cnt2 > TRIG, do_comp, lambda cth: cth,
                            (cnt2, th2))

        def scan_chunk(buf, chunk_base, start_grp, count, thr):
            def it(i, carry):
                cnt, th = carry
                base = i * (L * G)
                d = [buf[pl.ds(base + k * L, L)] for k in range(G)]
                ms = [dk <= th for dk in d]
                def ortree(xs):
                    while len(xs) > 1:
                        xs = [a | b for a, b in zip(xs[::2], xs[1::2])]
                    return xs[0]
                pc = plsc.all_reduce_population_count(ortree(ms))
                return lax.cond(
                    pc[0] > 0,
                    lambda c: group_append(c, d, ms, chunk_base + base),
                    lambda c: c, (cnt, th))

            return lax.fori_loop(start_grp, nv_u, it, (count, thr),
                                 unroll=2)

        def per_query(qi, carry):
            row = qbase + qi
            count = jnp.int32(0)
            thr = jnp.float32(INF)
            for c in range(nchunks):
                buf = bufs[c % 2]
                pltpu.make_async_copy(
                    d2_hbm.at[row, pl.ds(c * chunk, chunk)], buf,
                    sems[c % 2]).wait()
                if c < nchunks - 1:
                    pltpu.async_copy(
                        d2_hbm.at[row, pl.ds((c + 1) * chunk, chunk)],
                        bufs[(c + 1) % 2], sems[(c + 1) % 2])
                else:
                    @pl.when(qi + 1 < nq)
                    def _():
                        pltpu.async_copy(
                            d2_hbm.at[row + 1, pl.ds(0, chunk)], bufs[0],
                            sems[0])
                if c == 0:
                    for j in range(2 * G):
                        cval[pl.ds(j * L, L)] = buf[pl.ds(j * L, L)]
                        cidx[pl.ds(j * L, L)] = io + (j * L)
                    ts0 = _compact(cval, cidx, jnp.int32(2 * G * L))
                    write_back(ts0)
                    count = jnp.int32(TOPW)
                    thr = _lane(ts0[6], 1)
                    count, thr = scan_chunk(buf, 0, 2, count, thr)
                else:
                    count, thr = scan_chunk(buf, c * chunk, 0, count, thr)

            ts = _compact(cval, cidx, count)
            ts = _tie_fix(ts)
            for j in range(4):
                sval[pl.ds(j * L, L)] = ts[2 * j]
                sidx[pl.ds(j * L, L)] = ts[2 * j + 1]
            pltpu.sync_copy(sval, odist_hbm.at[row])
            pltpu.sync_copy(sidx, oidx_hbm.at[row])

            pltpu.async_copy(aug_hbm.at[sidx], rows, semg).wait()

            qk = [qkeys[qi, pl.ds(k * L, L)] for k in range(4)]

            def wbody(r, c):
                wsumv, vsumv = c
                acc = jnp.zeros((L,), jnp.float32)
                for k in range(4):
                    dk = rows[r, pl.ds(k * L, L)] - qk[k]
                    acc = acc + dk * dk
                wv = 1.0 / (_hsum16v(acc, io) + jnp.float32(DELTA))
                valv = _gather16(rows[r, pl.ds(DIM, L)], io * 0)
                return (wsumv + wv, vsumv + wv * valv)

            zv = jnp.zeros((L,), jnp.float32)
            wsumv, vsumv = lax.fori_loop(0, K, wbody, (zv, zv), unroll=False)
            oval = vsumv / wsumv
            nh = nq // L if nq >= L else 1
            for h in range(nh):
                @pl.when((qi // L) == h)
                def _():
                    cur = ovalbuf[pl.ds(h * L, L)]
                    ovalbuf[pl.ds(h * L, L)] = jnp.where(
                        io == (qi - h * L), oval, cur)
            return carry

        lax.fori_loop(0, nq, per_query, None, unroll=False)
        pltpu.sync_copy(ovalbuf, oval_hbm.at[pl.ds(qbase, nq)])

    return pl.kernel(
        body,
        out_type=[
            jax.ShapeDtypeStruct((batch,), jnp.float32),
            jax.ShapeDtypeStruct((batch, TOPW), jnp.int32),
            jax.ShapeDtypeStruct((batch, TOPW), jnp.float32),
        ],
        mesh=mesh,
        scratch_types=[
            pltpu.VMEM((chunk,), jnp.float32),
            pltpu.VMEM((chunk,), jnp.float32),
            pltpu.VMEM((nq, DIM), jnp.float32),
            pltpu.VMEM((CC,), jnp.float32),
            pltpu.VMEM((CC,), jnp.int32),
            pltpu.VMEM((TOPW, 2 * DIM), jnp.float32),
            pltpu.VMEM((TOPW,), jnp.int32),
            pltpu.VMEM((TOPW,), jnp.float32),
            pltpu.VMEM((max(nq, L),), jnp.float32),
            pltpu.SemaphoreType.DMA,
            pltpu.SemaphoreType.DMA,
            pltpu.SemaphoreType.DMA,
        ],
        compiler_params=pltpu.CompilerParams(needs_layout_passes=False),
        interpret=interpret,
    )


def kernel(key, keys_table, values_table):
    qsq = jnp.sum(key * key, axis=1, keepdims=True)
    kt_pad = jnp.pad(keys_table, ((0, CAP_PAD - CAP), (0, 0)))
    ksq_pad = jnp.pad(jnp.sum(keys_table * keys_table, axis=1),
                      (0, CAP_PAD - CAP), constant_values=1e30)[None, :]
    d2 = _d2_matrix(key, qsq, kt_pad, ksq_pad)
    aug = jnp.concatenate(
        [keys_table, values_table[:, None],
         jnp.zeros((CAP, 2 * DIM - DIM - 1), jnp.float32)], axis=1)
    nq = BATCH // NWORKERS
    sc = _sc_topk_combine(BATCH, CAP_PAD, CAP_PAD // 4, NWORKERS, nq, CAP)
    oval, oidx, odist = sc(d2, key, aug)
    return oval[:, None], oidx[:, :K], odist[:, :K]


# speedup vs baseline: 11.2266x; 1.0004x over previous
import numpy as np
import jax
import jax.numpy as jnp
from jax import lax
from jax.experimental import pallas as pl
from jax.experimental.pallas import tpu as pltpu
from jax.experimental.pallas import tpu_sc as plsc

DIM = 64
CAP = 100000
BATCH = 1024
K = 50
DELTA = 0.001

BC = 2048
CAP_PAD = ((CAP + BC - 1) // BC) * BC

L = 16
NWORKERS = 32
TOPW = 64
G = 16
CC = 672
TRIG = CC - G * L
INF = np.float32(3e38)
IMAX = np.int32(0x7FFFFFFF)


def _d2_block_kernel(q_ref, qsq_ref, kt_ref, ksq_ref, out_ref):
    qk = jax.lax.dot_general(
        q_ref[...], kt_ref[...],
        dimension_numbers=(((1,), (1,)), ((), ())),
        preferred_element_type=jnp.float32,
    )
    out_ref[...] = (qsq_ref[...] - 2.0 * qk) + ksq_ref[...]


def _d2_matrix(q, qsq, kt_pad, ksq_pad):
    grid = (CAP_PAD // BC,)
    return pl.pallas_call(
        _d2_block_kernel,
        grid=grid,
        in_specs=[
            pl.BlockSpec((BATCH, DIM), lambda i: (0, 0)),
            pl.BlockSpec((BATCH, 1), lambda i: (0, 0)),
            pl.BlockSpec((BC, DIM), lambda i: (i, 0)),
            pl.BlockSpec((1, BC), lambda i: (0, i)),
        ],
        out_specs=pl.BlockSpec((BATCH, BC), lambda i: (0, i)),
        out_shape=jax.ShapeDtypeStruct((BATCH, CAP_PAD), jnp.float32),
    )(q, qsq, kt_pad, ksq_pad)


def _merge2(av, ai, bv, bi):
    rv = lax.rev(bv, (0,))
    ri = lax.rev(bi, (0,))
    m = rv < av
    lov = jnp.where(m, rv, av)
    loi = jnp.where(m, ri, ai)
    hiv = jnp.where(m, av, rv)
    hii = jnp.where(m, ai, ri)
    lov, loi = plsc.sort_key_val(lov, loi)
    hiv, hii = plsc.sort_key_val(hiv, hii)
    return lov, loi, hiv, hii


def _insert(ts, sv, si):
    t0v, t0i, t1v, t1i, t2v, t2i, t3v, t3i = ts
    t0v, t0i, sv, si = _merge2(t0v, t0i, sv, si)
    t1v, t1i, sv, si = _merge2(t1v, t1i, sv, si)
    t2v, t2i, sv, si = _merge2(t2v, t2i, sv, si)
    t3v, t3i, sv, si = _merge2(t3v, t3i, sv, si)
    return (t0v, t0i, t1v, t1i, t2v, t2i, t3v, t3i)


def _compact(cval, cidx, count):
    io = lax.iota(jnp.int32, L)
    init = (jnp.full((L,), INF, jnp.float32), jnp.full((L,), IMAX, jnp.int32)) * 4

    def body(j, ts):
        base = j * L
        v = cval[pl.ds(base, L)]
        i = cidx[pl.ds(base, L)]
        valid = (base + io) < count
        v = jnp.where(valid, v, INF)
        i = jnp.where(valid, i, IMAX)
        sv, si = plsc.sort_key_val(v, i)
        return _insert(ts, sv, si)

    return lax.fori_loop(0, CC // L, body, init, unroll=False)


def _append(cval, cidx, count, d, idxv, m):
    n = _popcount(m)
    plsc.store_compressed(cval.at[pl.ds(count, L)], d, mask=m)
    plsc.store_compressed(cidx.at[pl.ds(count, L)], idxv, mask=m)
    return count + n


def _gather16(v, idx):
    dn = lax.GatherDimensionNumbers(offset_dims=(), collapsed_slice_dims=(0,),
                                    start_index_map=(0,))
    return lax.gather(v, idx[:, None], dn, (1,),
                      mode=lax.GatherScatterMode.PROMISE_IN_BOUNDS)


def _shift_up(v, boundary, io):
    g = _gather16(v, jnp.maximum(io - 1, 0))
    return jnp.where(io == 0, boundary, g)


def _shift_down(v, boundary, io):
    g = _gather16(v, jnp.minimum(io + 1, L - 1))
    return jnp.where(io == L - 1, boundary, g)


def _lane(v, lane):
    return v[lane]


def _popcount(m):
    return plsc.all_reduce_population_count(m)[0]


def _hsum16v(v, io):
    for sh in (8, 4, 2, 1):
        v = v + _gather16(v, (io + sh) & (L - 1))
    return v


def _tie_fix(ts):
    io = lax.iota(jnp.int32, L)
    lane_par = io % 2
    vs = list(ts)
    for p in (0, 1, 0, 1):
        t0v, t0i, t1v, t1i, t2v, t2i, t3v, t3i = vs
        tv = [t0v, t1v, t2v, t3v]
        ti = [t0i, t1i, t2i, t3i]
        new_i = []
        for j in range(4):
            pv_b = jnp.float32(-INF) if j == 0 else _lane(tv[j - 1], L - 1)
            pi_b = IMAX if j == 0 else _lane(ti[j - 1], L - 1)
            nv_b = jnp.float32(INF) if j == 3 else _lane(tv[j + 1], 0)
            ni_b = IMAX if j == 3 else _lane(ti[j + 1], 0)
            pv = _shift_up(tv[j], pv_b, io)
            pi = _shift_up(ti[j], pi_b, io)
            nv = _shift_down(tv[j], nv_b, io)
            ni = _shift_down(ti[j], ni_b, io)
            swap_prev = (tv[j] == pv) & (ti[j] < pi) & ((1 - lane_par) == p)
            swap_next = (nv == tv[j]) & (ni < ti[j]) & (lane_par == p)
            new_i.append(jnp.where(swap_prev, pi, jnp.where(swap_next, ni, ti[j])))
        vs = [t0v, new_i[0], t1v, new_i[1], t2v, new_i[2], t3v, new_i[3]]
    return tuple(vs)


def _sc_topk_combine(batch, cap_pad, chunk, nq):
    nchunks = cap_pad // chunk
    nv_u = chunk // (L * G)
    mesh = plsc.VectorSubcoreMesh(core_axis_name="c", subcore_axis_name="s",
                                  num_cores=2, num_subcores=16)

    def body(d2_hbm, key_hbm, aug_hbm,
             oval_hbm, oidx_hbm, odist_hbm,
             buf0, buf1, qkeys, cval, cidx, rows, sidx, sval, ovalbuf,
             sem0, sem1, semg):
        io = lax.iota(jnp.int32, L)
        wid = lax.axis_index("s") * 2 + lax.axis_index("c")
        qbase = wid * nq
        bufs = (buf0, buf1)
        sems = (sem0, sem1)

        pltpu.sync_copy(key_hbm.at[pl.ds(qbase, nq)], qkeys)
        pltpu.async_copy(d2_hbm.at[qbase, pl.ds(0, chunk)], buf0, sem0)

        def write_back(ts):
            for j in range(4):
                cval[pl.ds(j * L, L)] = ts[2 * j]
                cidx[pl.ds(j * L, L)] = ts[2 * j + 1]

        def group_append(c, d, ms, gbase):
            cnt2, th2 = c
            for k in range(G):
                idxv = io + (gbase + k * L)
                cnt2 = _append(cval, cidx, cnt2, d[k], idxv, ms[k])

            def do_comp(cth):
                ts = _compact(cval, cidx, cth[0])
                write_back(ts)
                return (jnp.int32(TOPW), _lane(ts[6], 1))

            return lax.cond(cnt2 > TRIG, do_comp, lambda cth: cth,
                            (cnt2, th2))

        def scan_chunk(buf, chunk_base, start_grp, count, thr):
            def it(i, carry):
                cnt, th = carry
                base = i * (L * G)
                d = [buf[pl.ds(base + k * L, L)] for k in range(G)]
                ms = [dk <= th for dk in d]
                def ortree(xs):
                    while len(xs) > 1:
                        xs = [a | b for a, b in zip(xs[::2], xs[1::2])]
                    return xs[0]
                pc = plsc.all_reduce_population_count(ortree(ms))
                return lax.cond(
                    pc[0] > 0,
                    lambda c: group_append(c, d, ms, chunk_base + base),
                    lambda c: c, (cnt, th))

            return lax.fori_loop(start_grp, nv_u, it, (count, thr),
                                 unroll=2)

        def per_query(qi, carry):
            row = qbase + qi
            count = jnp.int32(0)
            thr = jnp.float32(INF)
            for c in range(nchunks):
                buf = bufs[c % 2]
                pltpu.make_async_copy(
                    d2_hbm.at[row, pl.ds(c * chunk, chunk)], buf,
                    sems[c % 2]).wait()
                if c < nchunks - 1:
                    pltpu.async_copy(
                        d2_hbm.at[row, pl.ds((c + 1) * chunk, chunk)],
                        bufs[(c + 1) % 2], sems[(c + 1) % 2])
                else:
                    @pl.when(qi + 1 < nq)
                    def _():
                        pltpu.async_copy(
                            d2_hbm.at[row + 1, pl.ds(0, chunk)], bufs[0],
                            sems[0])
                if c == 0:
                    for j in range(2 * G):
                        cval[pl.ds(j * L, L)] = buf[pl.ds(j * L, L)]
                        cidx[pl.ds(j * L, L)] = io + (j * L)
                    ts0 = _compact(cval, cidx, jnp.int32(2 * G * L))
                    write_back(ts0)
                    count = jnp.int32(TOPW)
                    thr = _lane(ts0[6], 1)
                    count, thr = scan_chunk(buf, 0, 2, count, thr)
                else:
                    count, thr = scan_chunk(buf, c * chunk, 0, count, thr)

            ts = _compact(cval, cidx, count)
            ts = _tie_fix(ts)
            for j in range(4):
                sval[pl.ds(j * L, L)] = ts[2 * j]
                sidx[pl.ds(j * L, L)] = ts[2 * j + 1]
            pltpu.sync_copy(sval, odist_hbm.at[row])
            pltpu.sync_copy(sidx, oidx_hbm.at[row])

            pltpu.async_copy(aug_hbm.at[sidx], rows, semg).wait()

            qk = [qkeys[qi, pl.ds(k * L, L)] for k in range(4)]

            def wbody(r, c):
                wsumv, vsumv = c
                acc = jnp.zeros((L,), jnp.float32)
                for k in range(4):
                    dk = rows[r, pl.ds(k * L, L)] - qk[k]
                    acc = acc + dk * dk
                wv = 1.0 / (_hsum16v(acc, io) + jnp.float32(DELTA))
                valv = _gather16(rows[r, pl.ds(DIM, L)], io * 0)
                return (wsumv + wv, vsumv + wv * valv)

            zv = jnp.zeros((L,), jnp.float32)
            wsumv, vsumv = lax.fori_loop(0, K, wbody, (zv, zv), unroll=False)
            oval = vsumv / wsumv
            nh = nq // L if nq >= L else 1
            for h in range(nh):
                @pl.when((qi // L) == h)
                def _():
                    cur = ovalbuf[pl.ds(h * L, L)]
                    ovalbuf[pl.ds(h * L, L)] = jnp.where(
                        io == (qi - h * L), oval, cur)
            return carry

        lax.fori_loop(0, nq, per_query, None, unroll=False)
        pltpu.sync_copy(ovalbuf, oval_hbm.at[pl.ds(qbase, nq)])

    return pl.kernel(
        body,
        out_type=[
            jax.ShapeDtypeStruct((batch,), jnp.float32),
            jax.ShapeDtypeStruct((batch, TOPW), jnp.int32),
            jax.ShapeDtypeStruct((batch, TOPW), jnp.float32),
        ],
        mesh=mesh,
        scratch_types=[
            pltpu.VMEM((chunk,), jnp.float32),
            pltpu.VMEM((chunk,), jnp.float32),
            pltpu.VMEM((nq, DIM), jnp.float32),
            pltpu.VMEM((CC,), jnp.float32),
            pltpu.VMEM((CC,), jnp.int32),
            pltpu.VMEM((TOPW, 2 * DIM), jnp.float32),
            pltpu.VMEM((TOPW,), jnp.int32),
            pltpu.VMEM((TOPW,), jnp.float32),
            pltpu.VMEM((max(nq, L),), jnp.float32),
            pltpu.SemaphoreType.DMA,
            pltpu.SemaphoreType.DMA,
            pltpu.SemaphoreType.DMA,
        ],
        compiler_params=pltpu.CompilerParams(needs_layout_passes=False),
    )


def kernel(key, keys_table, values_table):
    qsq = jnp.sum(key * key, axis=1, keepdims=True)
    kt_pad = jnp.pad(keys_table, ((0, CAP_PAD - CAP), (0, 0)))
    ksq_pad = jnp.pad(jnp.sum(keys_table * keys_table, axis=1),
                      (0, CAP_PAD - CAP), constant_values=1e30)[None, :]
    d2 = _d2_matrix(key, qsq, kt_pad, ksq_pad)
    aug = jnp.concatenate(
        [keys_table, values_table[:, None],
         jnp.zeros((CAP, 2 * DIM - DIM - 1), jnp.float32)], axis=1)
    nq = BATCH // NWORKERS
    sc = _sc_topk_combine(BATCH, CAP_PAD, CAP_PAD // 4, nq)
    oval, oidx, odist = sc(d2, key, aug)
    return oval[:, None], oidx[:, :K], odist[:, :K]
